# baseline (speedup 1.0000x reference)
import functools

import jax
import jax.numpy as jnp
from jax.experimental import pallas as pl
from jax.experimental.pallas import tpu as pltpu

_BM = 1024
_NBUF = 3


def _router_body(x_hbm, w_ref, o_hbm, xbuf, obuf, in_sems, out_sems,
                 *, nsteps):
    def in_copy(s, buf):
        return pltpu.make_async_copy(
            x_hbm.at[pl.ds(s * _BM, _BM), :],
            xbuf.at[buf],
            in_sems.at[buf],
        )

    def out_copy(s, buf):
        return pltpu.make_async_copy(
            obuf.at[buf],
            o_hbm.at[pl.ds(s * _BM, _BM), :],
            out_sems.at[buf],
        )

    for i in range(_NBUF):
        in_copy(i, i).start()

    def step(s, carry):
        buf = jax.lax.rem(s, _NBUF)
        in_copy(s, buf).wait()

        @pl.when(s >= _NBUF)
        def _():
            out_copy(s - _NBUF, buf).wait()

        obuf[buf] = jnp.dot(xbuf[buf], w_ref[...],
                            preferred_element_type=jnp.float32)
        out_copy(s, buf).start()

        @pl.when(s + _NBUF < nsteps)
        def _():
            in_copy(s + _NBUF, buf).start()

        return carry

    jax.lax.fori_loop(0, nsteps, step, 0)

    for i in range(_NBUF):
        out_copy(nsteps - _NBUF + i, i).wait()


def kernel(hidden_states, W):
    M, K = hidden_states.shape
    K2, N = W.shape
    assert K == K2 and M % _BM == 0
    nsteps = M // _BM
    return pl.pallas_call(
        functools.partial(_router_body, nsteps=nsteps),
        in_specs=[
            pl.BlockSpec(memory_space=pl.ANY),
            pl.BlockSpec((K, N), lambda: (0, 0)),
        ],
        out_specs=pl.BlockSpec(memory_space=pl.ANY),
        out_shape=jax.ShapeDtypeStruct((M, N), jnp.float32),
        scratch_shapes=[
            pltpu.VMEM((_NBUF, _BM, K), jnp.float32),
            pltpu.VMEM((_NBUF, _BM, N), jnp.float32),
            pltpu.SemaphoreType.DMA((_NBUF,)),
            pltpu.SemaphoreType.DMA((_NBUF,)),
        ],
    )(hidden_states, W)


# speedup vs baseline: 1.0327x; 1.0327x over previous
import jax
import jax.numpy as jnp
from jax.experimental import pallas as pl
from jax.experimental.pallas import tpu as pltpu


def _router_matmul_kernel(w_hbm, x_ref, o_ref, w_vmem, w_sem):
    @pl.when(pl.program_id(0) == 0)
    def _():
        copy = pltpu.make_async_copy(w_hbm, w_vmem, w_sem)
        copy.start()
        copy.wait()

    o_ref[...] = jnp.dot(x_ref[...], w_vmem[...],
                         preferred_element_type=jnp.float32)


def kernel(hidden_states, W):
    M, K = hidden_states.shape
    K2, N = W.shape
    assert K == K2
    BM = 512
    grid = (M // BM,)
    return pl.pallas_call(
        _router_matmul_kernel,
        grid=grid,
        in_specs=[
            pl.BlockSpec(memory_space=pl.ANY),
            pl.BlockSpec((BM, K), lambda i: (i, 0)),
        ],
        out_specs=pl.BlockSpec((BM, N), lambda i: (i, 0)),
        out_shape=jax.ShapeDtypeStruct((M, N), jnp.float32),
        scratch_shapes=[
            pltpu.VMEM((K, N), jnp.float32),
            pltpu.SemaphoreType.DMA,
        ],
        compiler_params=pltpu.CompilerParams(
            dimension_semantics=("arbitrary",),
        ),
    )(W, hidden_states)


# speedup vs baseline: 1.0380x; 1.0052x over previous
import jax
import jax.numpy as jnp
from jax.experimental import pallas as pl
from jax.experimental.pallas import tpu as pltpu

_BM = 512


def _router_matmul_kernel(xa_ref, xb_ref, w_ref, o_ref):
    w = w_ref[...]
    o_ref[:_BM, :] = jnp.dot(xa_ref[...], w,
                             preferred_element_type=jnp.float32)
    o_ref[_BM:, :] = jnp.dot(xb_ref[...], w,
                             preferred_element_type=jnp.float32)


def kernel(hidden_states, W):
    M, K = hidden_states.shape
    K2, N = W.shape
    assert K == K2
    grid = (M // (2 * _BM),)
    return pl.pallas_call(
        _router_matmul_kernel,
        grid=grid,
        in_specs=[
            pl.BlockSpec((_BM, K), lambda i: (2 * i, 0)),
            pl.BlockSpec((_BM, K), lambda i: (2 * i + 1, 0)),
            pl.BlockSpec((K, N), lambda i: (0, 0)),
        ],
        out_specs=pl.BlockSpec((2 * _BM, N), lambda i: (i, 0)),
        out_shape=jax.ShapeDtypeStruct((M, N), jnp.float32),
        compiler_params=pltpu.CompilerParams(
            dimension_semantics=("parallel",),
        ),
    )(hidden_states, hidden_states, W)


# speedup vs baseline: 1.0458x; 1.0075x over previous
import functools

import jax
import jax.numpy as jnp
from jax.experimental import pallas as pl
from jax.experimental.pallas import tpu as pltpu

_BM = 512


def _router_body(x_hbm, w_ref, o_hbm, *, nsteps):
    def inner(x_blk, o_blk):
        o_blk[...] = jnp.dot(x_blk[...], w_ref[...],
                             preferred_element_type=jnp.float32)

    pltpu.emit_pipeline(
        inner,
        grid=(nsteps,),
        in_specs=[pl.BlockSpec((_BM, x_hbm.shape[1]), lambda i: (i, 0))],
        out_specs=[pl.BlockSpec((_BM, o_hbm.shape[1]), lambda i: (i, 0))],
    )(x_hbm, o_hbm)


def kernel(hidden_states, W):
    M, K = hidden_states.shape
    K2, N = W.shape
    assert K == K2 and M % _BM == 0
    return pl.pallas_call(
        functools.partial(_router_body, nsteps=M // _BM),
        in_specs=[
            pl.BlockSpec(memory_space=pl.ANY),
            pl.BlockSpec((K, N), lambda: (0, 0)),
        ],
        out_specs=pl.BlockSpec(memory_space=pl.ANY),
        out_shape=jax.ShapeDtypeStruct((M, N), jnp.float32),
    )(hidden_states, W)


# speedup vs baseline: 1.0506x; 1.0046x over previous
import jax
import jax.numpy as jnp
from jax.experimental import pallas as pl
from jax.experimental.pallas import tpu as pltpu

_BM = 512


def _router_matmul_kernel(x_ref, w_ref, o_ref):
    i = pl.program_id(0)
    o_ref[pl.ds(i * _BM, _BM), :] = jnp.dot(
        x_ref[...], w_ref[...], preferred_element_type=jnp.float32)


def kernel(hidden_states, W):
    M, K = hidden_states.shape
    K2, N = W.shape
    assert K == K2
    grid = (M // _BM,)
    return pl.pallas_call(
        _router_matmul_kernel,
        grid=grid,
        in_specs=[
            pl.BlockSpec((_BM, K), lambda i: (i, 0)),
            pl.BlockSpec((K, N), lambda i: (0, 0)),
        ],
        out_specs=pl.BlockSpec((M, N), lambda i: (0, 0)),
        out_shape=jax.ShapeDtypeStruct((M, N), jnp.float32),
        compiler_params=pltpu.CompilerParams(
            dimension_semantics=("arbitrary",),
        ),
    )(hidden_states, W)
